# baseline (speedup 1.0000x reference)
import functools
import math

import jax
import jax.numpy as jnp
from jax import lax
from jax.experimental import pallas as pl
from jax.experimental.pallas import tpu as pltpu
from jax.experimental.pallas import tpu_sc as plsc

D_MODEL = 64
SCALE = math.sqrt(D_MODEL)
NC = 2
NS = 16
NW = NC * NS


def _make_lookup(R, S, C, NB, K):
    rows_per_w = R // NW
    splits = S // C
    n_chunks = rows_per_w * splits
    assert rows_per_w * NW == R and splits * C == S
    assert C % 8 == 0 and 0 < K < NB and n_chunks % NB == 0
    assert n_chunks > 2 * NB

    mesh = plsc.VectorSubcoreMesh(core_axis_name="c", subcore_axis_name="s")

    scratch = [
        pltpu.VMEM((rows_per_w, S), jnp.int32),
    ] + [pltpu.VMEM((NB, C, 16), jnp.float32) for _ in range(4)
    ] + [pltpu.SemaphoreType.DMA] * (2 * NB)

    @functools.partial(
        pl.kernel,
        mesh=mesh,
        out_type=jax.ShapeDtypeStruct((R, S, 128), jnp.float32),
        scratch_types=scratch,
        compiler_params=pltpu.CompilerParams(use_tc_tiling_on_sc=False),
    )
    def lookup(x_hbm, t0, t1, t2, t3, out_hbm, idx_v, r0, r1, r2, r3,
               *sems):
        tabs = (t0, t1, t2, t3)
        rings = (r0, r1, r2, r3)
        gsem = sems[:NB]
        osem = sems[NB:]
        wid = lax.axis_index("s") * NC + lax.axis_index("c")
        base_row = wid * rows_per_w

        pltpu.sync_copy(x_hbm.at[pl.ds(base_row, rows_per_w)], idx_v)

        def fire_gather(g, b):
            r = g // splits
            h = g % splits
            start = pl.multiple_of(h * C, 8)
            idx = idx_v.at[r, pl.ds(start, C)]
            for k in range(4):
                pltpu.async_copy(tabs[k].at[idx], rings[k].at[b], gsem[b])

        def wait_gather(b):
            for k in range(4):
                pltpu.make_async_copy(
                    tabs[k].at[pl.ds(0, C)], rings[k].at[b], gsem[b]).wait()

        def fire_write(g, b):
            r = g // splits
            h = g % splits
            for k in range(4):
                dst = out_hbm.at[base_row + r,
                                 pl.ds(pl.multiple_of(h * C, 8), C),
                                 pl.ds(16 * k, 16)]
                pltpu.async_copy(rings[k].at[b], dst, osem[b])

        def wait_write(b):
            for k in range(4):
                pltpu.make_async_copy(
                    rings[k].at[b],
                    out_hbm.at[0, pl.ds(0, C), pl.ds(0, 16)],
                    osem[b]).wait()

        def scale(b):
            @pl.loop(0, C, unroll=4)
            def _(i):
                for k in range(4):
                    dst = rings[k].at[b]
                    s = pl.ds(0, 16)
                    dst[i, s] = dst[i, s] * SCALE

        for f in range(K):
            fire_gather(f, f)

        for g in range(NB - K):
            wait_gather(g % NB)
            scale(g % NB)
            fire_write(g, g % NB)
            fire_gather(g + K, (g + K) % NB)

        @pl.loop(NB - K, n_chunks - K, step=NB)
        def _(G):
            for db in range(NB):
                g = G + db
                b = (NB - K + db) % NB
                wait_gather(b)
                scale(b)
                fire_write(g, b)
                bf = (b + K) % NB
                wait_write(bf)
                fire_gather(g + K, bf)

        for dg in range(K):
            b = (NB - K + dg) % NB
            wait_gather(b)
            scale(b)
            fire_write(n_chunks - K + dg, b)
        for b in range(NB):
            wait_write(b)

    return lookup


def kernel(x, table):
    R, S = x.shape
    idx = x.astype(jnp.int32)
    parts = [table[:, 16 * k:16 * (k + 1)] for k in range(4)]
    out = _make_lookup(R, S, 40, 8, 4)(idx, *parts)
    return out[:, :, :D_MODEL]


# speedup vs baseline: 3.1987x; 3.1987x over previous
import functools
import math

import jax
import jax.numpy as jnp
from jax import lax
from jax.experimental import pallas as pl
from jax.experimental.pallas import tpu as pltpu
from jax.experimental.pallas import tpu_sc as plsc

D_MODEL = 64
SCALE = math.sqrt(D_MODEL)
NC = 2
NS = 16
NW = NC * NS


def _make_lookup(R, S, C, NB, K):
    rows_per_w = R // NW
    splits = S // C
    n_chunks = rows_per_w * splits
    assert rows_per_w * NW == R and splits * C == S
    assert C % 8 == 0 and 0 < K < NB and n_chunks % NB == 0
    assert n_chunks > 2 * NB

    mesh = plsc.VectorSubcoreMesh(core_axis_name="c", subcore_axis_name="s")

    scratch = [
        pltpu.VMEM((rows_per_w, S), jnp.int32),
        pltpu.VMEM((NB, C, D_MODEL), jnp.float32),
    ] + [pltpu.SemaphoreType.DMA] * (2 * NB)

    @functools.partial(
        pl.kernel,
        mesh=mesh,
        out_type=jax.ShapeDtypeStruct((R, S, 128), jnp.float32),
        scratch_types=scratch,
        compiler_params=pltpu.CompilerParams(use_tc_tiling_on_sc=False),
    )
    def lookup(x_hbm, table_hbm, out_hbm, idx_v, ring, *sems):
        gsem = sems[:NB]
        osem = sems[NB:]
        wid = lax.axis_index("s") * NC + lax.axis_index("c")
        base_row = wid * rows_per_w

        pltpu.sync_copy(x_hbm.at[pl.ds(base_row, rows_per_w)], idx_v)

        def fire_gather(g, b):
            r = g // splits
            h = g % splits
            start = pl.multiple_of(h * C, 8)
            idx = idx_v.at[r, pl.ds(start, C)]
            pltpu.async_copy(table_hbm.at[idx], ring.at[b], gsem[b])

        def wait_gather(b):
            pltpu.make_async_copy(
                table_hbm.at[pl.ds(0, C)], ring.at[b], gsem[b]).wait()

        def fire_write(g, b):
            r = g // splits
            h = g % splits
            dst = out_hbm.at[base_row + r, pl.ds(pl.multiple_of(h * C, 8), C),
                             pl.ds(0, D_MODEL)]
            pltpu.async_copy(ring.at[b], dst, osem[b])

        def wait_write(b):
            pltpu.make_async_copy(
                ring.at[b], out_hbm.at[0, pl.ds(0, C), pl.ds(0, D_MODEL)],
                osem[b]).wait()

        def scale(b):
            dst = ring.at[b]

            @pl.loop(0, C, unroll=4)
            def _(i):
                for j in range(D_MODEL // 16):
                    s = pl.ds(j * 16, 16)
                    dst[i, s] = dst[i, s] * SCALE

        for f in range(K):
            fire_gather(f, f)

        for g in range(NB - K):
            wait_gather(g % NB)
            scale(g % NB)
            fire_write(g, g % NB)
            fire_gather(g + K, (g + K) % NB)

        @pl.loop(NB - K, n_chunks - K, step=NB)
        def _(G):
            for db in range(NB):
                g = G + db
                b = (NB - K + db) % NB
                wait_gather(b)
                scale(b)
                fire_write(g, b)
                bf = (b + K) % NB
                wait_write(bf)
                fire_gather(g + K, bf)

        for dg in range(K):
            b = (NB - K + dg) % NB
            wait_gather(b)
            scale(b)
            fire_write(n_chunks - K + dg, b)
        for b in range(NB):
            wait_write(b)

    return lookup


def kernel(x, table):
    R, S = x.shape
    idx = x.astype(jnp.int32)
    out = _make_lookup(R, S, 40, 8, 4)(idx, table)
    return out[:, :, :D_MODEL]
